# Initial kernel scaffold; baseline (speedup 1.0000x reference)
import jax
import jax.numpy as jnp
from jax.experimental import pallas as pl


def kernel(deltas, class_logits, proposals):
    raise NotImplementedError("write your pallas kernel here")


# speedup vs baseline: 165.3429x; 165.3429x over previous
import jax
import jax.numpy as jnp
from jax.experimental import pallas as pl
from jax.experimental.pallas import tpu as pltpu

_IOU_THRESHOLD = 0.5
_SCORE_THRESHOLD = 0.05
_OUTPUT_BOX_NUM = 300


def _nms_body(dT_ref, lT_ref, lrows_ref, pT_ref, bo_ref, so_ref, lo_ref,
              dead_ref):
    C, N = lT_ref.shape[1], lT_ref.shape[2]
    K = bo_ref.shape[1]
    neg = jnp.float32(-1e30)

    lt = lT_ref[0]
    riota = jax.lax.broadcasted_iota(jnp.int32, (C, N), 0)
    M = jnp.max(lt, axis=0, keepdims=True)
    Z = jnp.sum(jnp.exp(lt - M), axis=0, keepdims=True)
    fgm = riota >= 1
    maxfg = jnp.max(jnp.where(fgm, lt, neg), axis=0, keepdims=True)
    best = jnp.min(
        jnp.where(jnp.logical_and(fgm, lt == maxfg), riota, C),
        axis=0, keepdims=True)
    fg = jnp.exp(maxfg - M) / Z

    onehot = riota == best
    d0 = jnp.sum(jnp.where(onehot, dT_ref[0, 0], 0.0), axis=0, keepdims=True)
    d1 = jnp.sum(jnp.where(onehot, dT_ref[0, 1], 0.0), axis=0, keepdims=True)
    d2 = jnp.sum(jnp.where(onehot, dT_ref[0, 2], 0.0), axis=0, keepdims=True)
    d3 = jnp.sum(jnp.where(onehot, dT_ref[0, 3], 0.0), axis=0, keepdims=True)

    p = pT_ref[0]
    y1p, x1p, y2p, x2p = p[0:1], p[1:2], p[2:3], p[3:4]
    h = y2p - y1p
    w = x2p - x1p
    cy = (y2p + y1p) * 0.5 + (d0 * 0.1) * h
    cx = (x2p + x1p) * 0.5 + (d1 * 0.1) * w
    hh = h * jnp.exp(d2 * 0.2)
    ww = w * jnp.exp(d3 * 0.2)
    y1 = cy - hh * 0.5
    x1 = cx - ww * 0.5
    y2 = cy + hh * 0.5
    x2 = cx + ww * 0.5
    area = jnp.maximum(y2 - y1, 0.0) * jnp.maximum(x2 - x1, 0.0)

    bo_ref[...] = jnp.zeros(bo_ref.shape, bo_ref.dtype)
    so_ref[...] = jnp.zeros(so_ref.shape, so_ref.dtype)
    lo_ref[...] = jnp.zeros(lo_ref.shape, lo_ref.dtype)

    laneN = jax.lax.broadcasted_iota(jnp.int32, (1, N), 1)
    valid = fg > _SCORE_THRESHOLD
    dead_ref[...] = jnp.zeros(dead_ref.shape, dead_ref.dtype)

    def round_body(k, _):
        dead = dead_ref[...] > 0.0
        cand = jnp.logical_and(valid, jnp.logical_not(dead))
        ms = jnp.where(cand, fg, neg)
        m = jnp.max(ms)
        active = m > _SCORE_THRESHOLD
        istar = jnp.min(jnp.where(jnp.logical_and(cand, ms == m), laneN, N))
        sel = laneN == istar
        y1s = jnp.sum(jnp.where(sel, y1, 0.0))
        x1s = jnp.sum(jnp.where(sel, x1, 0.0))
        y2s = jnp.sum(jnp.where(sel, y2, 0.0))
        x2s = jnp.sum(jnp.where(sel, x2, 0.0))
        area_s = jnp.maximum(y2s - y1s, 0.0) * jnp.maximum(x2s - x1s, 0.0)
        inter = (jnp.maximum(jnp.minimum(y2s, y2) - jnp.maximum(y1s, y1), 0.0)
                 * jnp.maximum(jnp.minimum(x2s, x2) - jnp.maximum(x1s, x1), 0.0))
        iou = inter / jnp.maximum(area_s + area - inter, 1e-8)
        kill = jnp.logical_or(iou > _IOU_THRESHOLD, sel)

        @pl.when(active)
        def _emit():
            li5 = jax.lax.broadcasted_iota(jnp.int32, (1, 5), 1)
            row5 = jnp.where(
                li5 == 0, y1s,
                jnp.where(li5 == 1, x1s,
                          jnp.where(li5 == 2, y2s,
                                    jnp.where(li5 == 3, x2s, 1.0))))
            bo_ref[0, pl.ds(k, 1), :] = row5
            li2 = jax.lax.broadcasted_iota(jnp.int32, (1, 2), 1)
            so_ref[0, pl.ds(k, 1), :] = jnp.where(li2 == 0, m, 1.0)
            lo_ref[0, pl.ds(k, 1), 0:C] = lrows_ref[0, pl.ds(istar, 1), :]
            lo_ref[0, pl.ds(k, 1), C:C + 1] = jnp.full((1, 1), 1.0, jnp.float32)
            dead_ref[...] = jnp.where(jnp.logical_or(dead, kill), 1.0, 0.0)

        return 0

    jax.lax.fori_loop(0, K, round_body, 0)


def kernel(deltas, class_logits, proposals):
    B, N, C = class_logits.shape
    K = _OUTPUT_BOX_NUM
    deltasT = jnp.transpose(deltas, (0, 3, 2, 1))
    logitsT = jnp.transpose(class_logits, (0, 2, 1))
    propsT = jnp.transpose(proposals, (0, 2, 1))

    out_shape = [
        jax.ShapeDtypeStruct((B, K, 5), jnp.float32),
        jax.ShapeDtypeStruct((B, K, 2), jnp.float32),
        jax.ShapeDtypeStruct((B, K, C + 1), jnp.float32),
    ]
    boxes, scores, logits_out = pl.pallas_call(
        _nms_body,
        grid=(B,),
        in_specs=[
            pl.BlockSpec((1, 4, C, N), lambda b: (b, 0, 0, 0)),
            pl.BlockSpec((1, C, N), lambda b: (b, 0, 0)),
            pl.BlockSpec((1, N, C), lambda b: (b, 0, 0)),
            pl.BlockSpec((1, 5, N), lambda b: (b, 0, 0)),
        ],
        out_specs=[
            pl.BlockSpec((1, K, 5), lambda b: (b, 0, 0)),
            pl.BlockSpec((1, K, 2), lambda b: (b, 0, 0)),
            pl.BlockSpec((1, K, C + 1), lambda b: (b, 0, 0)),
        ],
        out_shape=out_shape,
        scratch_shapes=[pltpu.VMEM((1, N), jnp.float32)],
    )(deltasT, logitsT, class_logits, propsT)
    return boxes, scores, logits_out


# speedup vs baseline: 165.3926x; 1.0003x over previous
import jax
import jax.numpy as jnp
from jax.experimental import pallas as pl
from jax.experimental.pallas import tpu as pltpu

_IOU_THRESHOLD = 0.5
_SCORE_THRESHOLD = 0.05
_OUTPUT_BOX_NUM = 300


def _nms_body(dT_ref, lT_ref, lrows_ref, pT_ref, bo_ref, so_ref, lo_ref,
              dead_ref):
    C, N = lT_ref.shape[1], lT_ref.shape[2]
    K = bo_ref.shape[1]
    neg = jnp.float32(-1e30)

    lt = lT_ref[0]
    riota = jax.lax.broadcasted_iota(jnp.int32, (C, N), 0)
    M = jnp.max(lt, axis=0, keepdims=True)
    Z = jnp.sum(jnp.exp(lt - M), axis=0, keepdims=True)
    fgm = riota >= 1
    maxfg = jnp.max(jnp.where(fgm, lt, neg), axis=0, keepdims=True)
    best = jnp.min(
        jnp.where(jnp.logical_and(fgm, lt == maxfg), riota, C),
        axis=0, keepdims=True)
    fg = jnp.exp(maxfg - M) / Z

    onehot = riota == best
    d0 = jnp.sum(jnp.where(onehot, dT_ref[0, 0], 0.0), axis=0, keepdims=True)
    d1 = jnp.sum(jnp.where(onehot, dT_ref[0, 1], 0.0), axis=0, keepdims=True)
    d2 = jnp.sum(jnp.where(onehot, dT_ref[0, 2], 0.0), axis=0, keepdims=True)
    d3 = jnp.sum(jnp.where(onehot, dT_ref[0, 3], 0.0), axis=0, keepdims=True)

    p = pT_ref[0]
    y1p, x1p, y2p, x2p = p[0:1], p[1:2], p[2:3], p[3:4]
    h = y2p - y1p
    w = x2p - x1p
    cy = (y2p + y1p) * 0.5 + (d0 * 0.1) * h
    cx = (x2p + x1p) * 0.5 + (d1 * 0.1) * w
    hh = h * jnp.exp(d2 * 0.2)
    ww = w * jnp.exp(d3 * 0.2)
    y1 = cy - hh * 0.5
    x1 = cx - ww * 0.5
    y2 = cy + hh * 0.5
    x2 = cx + ww * 0.5
    area = jnp.maximum(y2 - y1, 0.0) * jnp.maximum(x2 - x1, 0.0)

    bo_ref[...] = jnp.zeros(bo_ref.shape, bo_ref.dtype)
    so_ref[...] = jnp.zeros(so_ref.shape, so_ref.dtype)
    lo_ref[...] = jnp.zeros(lo_ref.shape, lo_ref.dtype)

    laneN = jax.lax.broadcasted_iota(jnp.int32, (1, N), 1)
    valid = fg > _SCORE_THRESHOLD
    dead_ref[...] = jnp.zeros(dead_ref.shape, dead_ref.dtype)

    def round_body(k, _):
        dead = dead_ref[...] > 0.0
        cand = jnp.logical_and(valid, jnp.logical_not(dead))
        ms = jnp.where(cand, fg, neg)
        m = jnp.max(ms)
        active = m > _SCORE_THRESHOLD
        istar = jnp.min(jnp.where(jnp.logical_and(cand, ms == m), laneN, N))
        sel = laneN == istar
        y1s = jnp.sum(jnp.where(sel, y1, 0.0))
        x1s = jnp.sum(jnp.where(sel, x1, 0.0))
        y2s = jnp.sum(jnp.where(sel, y2, 0.0))
        x2s = jnp.sum(jnp.where(sel, x2, 0.0))
        area_s = jnp.maximum(y2s - y1s, 0.0) * jnp.maximum(x2s - x1s, 0.0)
        inter = (jnp.maximum(jnp.minimum(y2s, y2) - jnp.maximum(y1s, y1), 0.0)
                 * jnp.maximum(jnp.minimum(x2s, x2) - jnp.maximum(x1s, x1), 0.0))
        iou = inter / jnp.maximum(area_s + area - inter, 1e-8)
        kill = jnp.logical_or(iou > _IOU_THRESHOLD, sel)

        @pl.when(active)
        def _emit():
            li5 = jax.lax.broadcasted_iota(jnp.int32, (1, 5), 1)
            row5 = jnp.where(
                li5 == 0, y1s,
                jnp.where(li5 == 1, x1s,
                          jnp.where(li5 == 2, y2s,
                                    jnp.where(li5 == 3, x2s, 1.0))))
            bo_ref[0, pl.ds(k, 1), :] = row5
            li2 = jax.lax.broadcasted_iota(jnp.int32, (1, 2), 1)
            so_ref[0, pl.ds(k, 1), :] = jnp.where(li2 == 0, m, 1.0)
            lo_ref[0, pl.ds(k, 1), 0:C] = lrows_ref[0, pl.ds(istar, 1), :]
            lo_ref[0, pl.ds(k, 1), C:C + 1] = jnp.full((1, 1), 1.0, jnp.float32)
            dead_ref[...] = jnp.where(jnp.logical_or(dead, kill), 1.0, 0.0)

        return 0

    jax.lax.fori_loop(0, K, round_body, 0)


def kernel(deltas, class_logits, proposals):
    B, N, C = class_logits.shape
    K = _OUTPUT_BOX_NUM
    deltasT = jnp.transpose(deltas, (0, 3, 2, 1))
    logitsT = jnp.transpose(class_logits, (0, 2, 1))
    propsT = jnp.transpose(proposals, (0, 2, 1))

    out_shape = [
        jax.ShapeDtypeStruct((B, K, 5), jnp.float32),
        jax.ShapeDtypeStruct((B, K, 2), jnp.float32),
        jax.ShapeDtypeStruct((B, K, C + 1), jnp.float32),
    ]
    boxes, scores, logits_out = pl.pallas_call(
        _nms_body,
        grid=(B,),
        in_specs=[
            pl.BlockSpec((1, 4, C, N), lambda b: (b, 0, 0, 0)),
            pl.BlockSpec((1, C, N), lambda b: (b, 0, 0)),
            pl.BlockSpec((1, N, C), lambda b: (b, 0, 0)),
            pl.BlockSpec((1, 5, N), lambda b: (b, 0, 0)),
        ],
        out_specs=[
            pl.BlockSpec((1, K, 5), lambda b: (b, 0, 0)),
            pl.BlockSpec((1, K, 2), lambda b: (b, 0, 0)),
            pl.BlockSpec((1, K, C + 1), lambda b: (b, 0, 0)),
        ],
        out_shape=out_shape,
        scratch_shapes=[pltpu.VMEM((1, N), jnp.float32)],
        compiler_params=pltpu.CompilerParams(
            dimension_semantics=("parallel",)),
    )(deltasT, logitsT, class_logits, propsT)
    return boxes, scores, logits_out


# speedup vs baseline: 243.3280x; 1.4712x over previous
import jax
import jax.numpy as jnp
from jax.experimental import pallas as pl
from jax.experimental.pallas import tpu as pltpu

_IOU_THRESHOLD = 0.5
_SCORE_THRESHOLD = 0.05
_OUTPUT_BOX_NUM = 300


def _nms_body(dK_ref, lK_ref, lrows_ref, pK_ref, bo_ref, so_ref, lo_ref,
              ls_ref):
    C, R, Q = lK_ref.shape[1], lK_ref.shape[2], lK_ref.shape[3]
    N = R * Q
    K = bo_ref.shape[1]
    neg = jnp.float32(-1e30)

    lt = lK_ref[0]
    cio = jax.lax.broadcasted_iota(jnp.int32, (C, R, Q), 0)
    M = jnp.max(lt, axis=0)
    Z = jnp.sum(jnp.exp(lt - M[None]), axis=0)
    fgm = cio >= 1
    maxfg = jnp.max(jnp.where(fgm, lt, neg), axis=0)
    best = jnp.min(
        jnp.where(jnp.logical_and(fgm, lt == maxfg[None]), cio, C),
        axis=0)
    fg = jnp.exp(maxfg - M) / Z

    onehot = cio == best[None]
    d0 = jnp.sum(jnp.where(onehot, dK_ref[0, 0:C], 0.0), axis=0)
    d1 = jnp.sum(jnp.where(onehot, dK_ref[0, C:2 * C], 0.0), axis=0)
    d2 = jnp.sum(jnp.where(onehot, dK_ref[0, 2 * C:3 * C], 0.0), axis=0)
    d3 = jnp.sum(jnp.where(onehot, dK_ref[0, 3 * C:4 * C], 0.0), axis=0)

    y1p = pK_ref[0, 0]
    x1p = pK_ref[0, 1]
    y2p = pK_ref[0, 2]
    x2p = pK_ref[0, 3]
    h = y2p - y1p
    w = x2p - x1p
    cy = (y2p + y1p) * 0.5 + (d0 * 0.1) * h
    cx = (x2p + x1p) * 0.5 + (d1 * 0.1) * w
    hh = h * jnp.exp(d2 * 0.2)
    ww = w * jnp.exp(d3 * 0.2)
    y1 = cy - hh * 0.5
    x1 = cx - ww * 0.5
    y2 = cy + hh * 0.5
    x2 = cx + ww * 0.5
    area = jnp.maximum(y2 - y1, 0.0) * jnp.maximum(x2 - x1, 0.0)

    bo_ref[...] = jnp.zeros(bo_ref.shape, bo_ref.dtype)
    so_ref[...] = jnp.zeros(so_ref.shape, so_ref.dtype)
    lo_ref[...] = jnp.zeros(lo_ref.shape, lo_ref.dtype)

    fio = (jax.lax.broadcasted_iota(jnp.int32, (R, Q), 0) * Q
           + jax.lax.broadcasted_iota(jnp.int32, (R, Q), 1))
    ls_ref[...] = jnp.where(fg > _SCORE_THRESHOLD, fg, neg)

    def round_body(k, _):
        ls = ls_ref[...]
        m = jnp.max(ls)
        active = m > _SCORE_THRESHOLD
        istar = jnp.min(jnp.where(ls == m, fio, N))
        sel = fio == istar
        y1s = jnp.sum(jnp.where(sel, y1, 0.0))
        x1s = jnp.sum(jnp.where(sel, x1, 0.0))
        y2s = jnp.sum(jnp.where(sel, y2, 0.0))
        x2s = jnp.sum(jnp.where(sel, x2, 0.0))
        area_s = jnp.maximum(y2s - y1s, 0.0) * jnp.maximum(x2s - x1s, 0.0)
        inter = (jnp.maximum(jnp.minimum(y2s, y2) - jnp.maximum(y1s, y1), 0.0)
                 * jnp.maximum(jnp.minimum(x2s, x2) - jnp.maximum(x1s, x1), 0.0))
        iou = inter / jnp.maximum(area_s + area - inter, 1e-8)
        kill = jnp.logical_or(iou > _IOU_THRESHOLD, sel)

        @pl.when(active)
        def _emit():
            li5 = jax.lax.broadcasted_iota(jnp.int32, (1, 5), 1)
            row5 = jnp.where(
                li5 == 0, y1s,
                jnp.where(li5 == 1, x1s,
                          jnp.where(li5 == 2, y2s,
                                    jnp.where(li5 == 3, x2s, 1.0))))
            bo_ref[0, pl.ds(k, 1), :] = row5
            li2 = jax.lax.broadcasted_iota(jnp.int32, (1, 2), 1)
            so_ref[0, pl.ds(k, 1), :] = jnp.where(li2 == 0, m, 1.0)
            lo_ref[0, pl.ds(k, 1), 0:C] = lrows_ref[0, pl.ds(istar, 1), :]
            lo_ref[0, pl.ds(k, 1), C:C + 1] = jnp.full((1, 1), 1.0, jnp.float32)
            ls_ref[...] = jnp.where(kill, neg, ls)

        return 0

    jax.lax.fori_loop(0, K, round_body, 0)


def kernel(deltas, class_logits, proposals):
    B, N, C = class_logits.shape
    K = _OUTPUT_BOX_NUM
    R = 8
    Q = N // R
    assert N == R * Q
    deltasK = jnp.transpose(deltas, (0, 3, 2, 1)).reshape(B, 4 * C, R, Q)
    logitsK = jnp.transpose(class_logits, (0, 2, 1)).reshape(B, C, R, Q)
    propsK = jnp.transpose(proposals, (0, 2, 1)).reshape(B, 5, R, Q)

    out_shape = [
        jax.ShapeDtypeStruct((B, K, 5), jnp.float32),
        jax.ShapeDtypeStruct((B, K, 2), jnp.float32),
        jax.ShapeDtypeStruct((B, K, C + 1), jnp.float32),
    ]
    boxes, scores, logits_out = pl.pallas_call(
        _nms_body,
        grid=(B,),
        in_specs=[
            pl.BlockSpec((1, 4 * C, R, Q), lambda b: (b, 0, 0, 0)),
            pl.BlockSpec((1, C, R, Q), lambda b: (b, 0, 0, 0)),
            pl.BlockSpec((1, N, C), lambda b: (b, 0, 0)),
            pl.BlockSpec((1, 5, R, Q), lambda b: (b, 0, 0, 0)),
        ],
        out_specs=[
            pl.BlockSpec((1, K, 5), lambda b: (b, 0, 0)),
            pl.BlockSpec((1, K, 2), lambda b: (b, 0, 0)),
            pl.BlockSpec((1, K, C + 1), lambda b: (b, 0, 0)),
        ],
        out_shape=out_shape,
        scratch_shapes=[pltpu.VMEM((R, Q), jnp.float32)],
        compiler_params=pltpu.CompilerParams(
            dimension_semantics=("parallel",)),
    )(deltasK, logitsK, class_logits, propsK)
    return boxes, scores, logits_out


# speedup vs baseline: 473.3463x; 1.9453x over previous
import jax
import jax.numpy as jnp
from jax.experimental import pallas as pl

_IOU_THRESHOLD = 0.5
_SCORE_THRESHOLD = 0.05
_OUTPUT_BOX_NUM = 300


def _reduce11(x, op):
    x = op(x, axis=1, keepdims=True)
    return op(x, axis=0, keepdims=True)


def _nms_body(dK_ref, lK_ref, lrows_ref, pK_ref, bo_ref, so_ref, lo_ref):
    B, C = lK_ref.shape[0], lK_ref.shape[1]
    R, Q = lK_ref.shape[2], lK_ref.shape[3]
    N = R * Q
    K = bo_ref.shape[1]
    neg = jnp.float32(-1e30)

    cio = jax.lax.broadcasted_iota(jnp.int32, (C, R, Q), 0)
    fgm = cio >= 1
    fio = (jax.lax.broadcasted_iota(jnp.int32, (R, Q), 0) * Q
           + jax.lax.broadcasted_iota(jnp.int32, (R, Q), 1))
    li5 = jax.lax.broadcasted_iota(jnp.int32, (1, 5), 1)
    li2 = jax.lax.broadcasted_iota(jnp.int32, (1, 2), 1)

    per_img = []
    for b in range(B):
        lt = lK_ref[b]
        M = jnp.max(lt, axis=0)
        Z = jnp.sum(jnp.exp(lt - M[None]), axis=0)
        maxfg = jnp.max(jnp.where(fgm, lt, neg), axis=0)
        best = jnp.min(
            jnp.where(jnp.logical_and(fgm, lt == maxfg[None]), cio, C),
            axis=0)
        fg = jnp.exp(maxfg - M) / Z

        onehot = cio == best[None]
        d0 = jnp.sum(jnp.where(onehot, dK_ref[b, 0:C], 0.0), axis=0)
        d1 = jnp.sum(jnp.where(onehot, dK_ref[b, C:2 * C], 0.0), axis=0)
        d2 = jnp.sum(jnp.where(onehot, dK_ref[b, 2 * C:3 * C], 0.0), axis=0)
        d3 = jnp.sum(jnp.where(onehot, dK_ref[b, 3 * C:4 * C], 0.0), axis=0)

        y1p = pK_ref[b, 0]
        x1p = pK_ref[b, 1]
        y2p = pK_ref[b, 2]
        x2p = pK_ref[b, 3]
        h = y2p - y1p
        w = x2p - x1p
        cy = (y2p + y1p) * 0.5 + (d0 * 0.1) * h
        cx = (x2p + x1p) * 0.5 + (d1 * 0.1) * w
        hh = h * jnp.exp(d2 * 0.2)
        ww = w * jnp.exp(d3 * 0.2)
        y1 = cy - hh * 0.5
        x1 = cx - ww * 0.5
        y2 = cy + hh * 0.5
        x2 = cx + ww * 0.5
        area = jnp.maximum(y2 - y1, 0.0) * jnp.maximum(x2 - x1, 0.0)
        per_img.append((y1, x1, y2, x2, area,
                        jnp.where(fg > _SCORE_THRESHOLD, fg, neg)))

    def round_body(k, carry):
        new_ls = []
        for b in range(B):
            y1, x1, y2, x2, area, _ = per_img[b]
            ls = carry[b]
            mv = _reduce11(ls, jnp.max)
            actv = mv > _SCORE_THRESHOLD
            iv = _reduce11(jnp.where(ls == mv, fio, N), jnp.min)
            istar = iv[0, 0]
            sel = fio == iv
            y1v = _reduce11(jnp.where(sel, y1, neg), jnp.max)
            x1v = _reduce11(jnp.where(sel, x1, neg), jnp.max)
            y2v = _reduce11(jnp.where(sel, y2, neg), jnp.max)
            x2v = _reduce11(jnp.where(sel, x2, neg), jnp.max)
            area_v = (jnp.maximum(y2v - y1v, 0.0)
                      * jnp.maximum(x2v - x1v, 0.0))
            inter = (jnp.maximum(jnp.minimum(y2v, y2) - jnp.maximum(y1v, y1),
                                 0.0)
                     * jnp.maximum(jnp.minimum(x2v, x2) - jnp.maximum(x1v, x1),
                                   0.0))
            iou = inter / jnp.maximum(area_v + area - inter, 1e-8)
            kill = jnp.logical_or(iou > _IOU_THRESHOLD, sel)
            new_ls.append(
                jnp.where(jnp.logical_and(kill, actv), neg, ls))

            row5 = jnp.where(
                li5 == 0, y1v,
                jnp.where(li5 == 1, x1v,
                          jnp.where(li5 == 2, y2v,
                                    jnp.where(li5 == 3, x2v, 1.0))))
            bo_ref[b, pl.ds(k, 1), :] = jnp.where(actv, row5, 0.0)
            so_ref[b, pl.ds(k, 1), :] = jnp.where(
                actv, jnp.where(li2 == 0, mv, 1.0), 0.0)
            lrow = lrows_ref[b, pl.ds(istar, 1), :]
            lo_ref[b, pl.ds(k, 1), 0:C] = jnp.where(actv, lrow, 0.0)
            lo_ref[b, pl.ds(k, 1), C:C + 1] = jnp.where(actv, 1.0, 0.0)
        return tuple(new_ls)

    jax.lax.fori_loop(0, K, round_body,
                      tuple(p[5] for p in per_img))


def kernel(deltas, class_logits, proposals):
    B, N, C = class_logits.shape
    K = _OUTPUT_BOX_NUM
    R = 8
    Q = N // R
    assert N == R * Q
    deltasK = jnp.transpose(deltas, (0, 3, 2, 1)).reshape(B, 4 * C, R, Q)
    logitsK = jnp.transpose(class_logits, (0, 2, 1)).reshape(B, C, R, Q)
    propsK = jnp.transpose(proposals, (0, 2, 1)).reshape(B, 5, R, Q)

    out_shape = [
        jax.ShapeDtypeStruct((B, K, 5), jnp.float32),
        jax.ShapeDtypeStruct((B, K, 2), jnp.float32),
        jax.ShapeDtypeStruct((B, K, C + 1), jnp.float32),
    ]
    boxes, scores, logits_out = pl.pallas_call(
        _nms_body,
        out_shape=out_shape,
    )(deltasK, logitsK, class_logits, propsK)
    return boxes, scores, logits_out
